# baseline (speedup 1.0000x reference)
import functools

import jax
import jax.numpy as jnp
from jax import lax
from jax.experimental import pallas as pl
from jax.experimental.pallas import tpu as pltpu
from jax.experimental.pallas import tpu_sc as plsc

NUM_EMBEDDING = 1000000
EMBEDDING_DIM = 64
IBLK = 128
TILE = 8 * IBLK

_INFO = plsc.get_sparse_core_info()
_NC = _INFO.num_cores
_NS = _INFO.num_subcores
_NW = _NC * _NS


def _make_lookup(n_tokens, n_per):
    i_per_w = n_tokens // _NW
    nblk = i_per_w // IBLK
    n_iblk = n_tokens // IBLK
    n_jobs = n_per * nblk
    dtiles = EMBEDDING_DIM // 8
    mesh = plsc.VectorSubcoreMesh(core_axis_name="c", subcore_axis_name="s")

    @functools.partial(
        pl.kernel,
        mesh=mesh,
        out_type=jax.ShapeDtypeStruct((n_tokens * n_per * EMBEDDING_DIM,),
                                      jnp.float32),
        scratch_types=[
            pltpu.VMEM((n_per, i_per_w), jnp.int32),
            pltpu.VMEM((2, IBLK, EMBEDDING_DIM), jnp.float32),
            pltpu.VMEM((2, EMBEDDING_DIM * IBLK), jnp.float32),
            pltpu.SemaphoreType.DMA,
            pltpu.SemaphoreType.DMA,
        ],
        compiler_params=pltpu.CompilerParams(use_tc_tiling_on_sc=False,
                                             needs_layout_passes=False),
    )
    def lookup_kernel(tok_hbm, table_hbm, out_hbm, idx_v, rows_v, t_v,
                      gsem, wsem):
        wid = lax.axis_index("s") * _NC + lax.axis_index("c")
        i0w = wid * i_per_w
        pltpu.sync_copy(tok_hbm.at[:, pl.ds(i0w, i_per_w)], idx_v)
        lanes = lax.iota(jnp.int32, 16)

        def fire_gather(job, buf):
            j = lax.div(job, nblk)
            ib = lax.rem(job, nblk)
            pltpu.async_copy(
                table_hbm.at[idx_v.at[j, pl.ds(ib * IBLK, IBLK)]],
                rows_v.at[buf],
                gsem,
            )

        def drain_gather(buf):
            pltpu.make_async_copy(
                table_hbm.at[pl.ds(0, IBLK)],
                rows_v.at[buf],
                gsem,
            ).wait()

        def drain_writebacks():
            pltpu.make_async_copy(
                t_v.at[0],
                out_hbm.at[pl.ds(0, EMBEDDING_DIM * IBLK)],
                wsem,
            ).wait()

        def transpose_into(buf):
            def ic_body(ic, carry):
                tok16 = ic * 16 + lanes
                for d in range(EMBEDDING_DIM):
                    vals = plsc.load_gather(
                        rows_v.at[buf],
                        [tok16, jnp.full((16,), d, jnp.int32)],
                    )
                    t_v[buf, pl.ds(d * IBLK + ic * 16, 16)] = vals
                return carry

            lax.fori_loop(0, IBLK // 16, ic_body, 0)

        def fire_writebacks(job, buf):
            j = lax.div(job, nblk)
            ib = lax.rem(job, nblk)
            gblk = wid * nblk + ib
            for db in range(dtiles):
                pltpu.async_copy(
                    t_v.at[buf, pl.ds(db * TILE, TILE)],
                    out_hbm.at[pl.ds(((j * dtiles + db) * n_iblk + gblk)
                                     * TILE, TILE)],
                    wsem,
                )

        fire_gather(0, 0)
        fire_writebacks(0, 1)

        def job_body(job, carry):
            cur = lax.rem(job, 2)
            nxt = 1 - cur
            drain_gather(cur)
            nxt_job = lax.min(job + 1, n_jobs - 1)
            fire_gather(nxt_job, nxt)
            transpose_into(cur)
            drain_writebacks()
            fire_writebacks(job, cur)
            return carry

        lax.fori_loop(0, n_jobs, job_body, 0)
        drain_gather(lax.rem(n_jobs, 2))
        drain_writebacks()

    return lookup_kernel


def kernel(token_ids, weight):
    n_tokens, n_per = token_ids.shape
    tok2 = token_ids.T.astype(jnp.int32)
    out1d = _make_lookup(n_tokens, n_per)(tok2, weight)
    o5 = out1d.reshape(n_per, EMBEDDING_DIM // 8, n_tokens // IBLK, 8, IBLK)
    return o5.transpose(2, 4, 0, 1, 3).reshape(n_tokens, n_per, EMBEDDING_DIM)


# speedup vs baseline: 1.1941x; 1.1941x over previous
import functools

import jax
import jax.numpy as jnp
from jax import lax
from jax.experimental import pallas as pl
from jax.experimental.pallas import tpu as pltpu
from jax.experimental.pallas import tpu_sc as plsc

NUM_EMBEDDING = 1000000
EMBEDDING_DIM = 64
IBLK = 128
TILE = 8 * IBLK

_INFO = plsc.get_sparse_core_info()
_NC = _INFO.num_cores
_NS = _INFO.num_subcores
_NW = _NC * _NS


def _make_lookup(n_tokens, n_per):
    i_per_w = n_tokens // _NW
    nblk = i_per_w // IBLK
    n_iblk = n_tokens // IBLK
    n_jobs = n_per * nblk
    dtiles = EMBEDDING_DIM // 8
    mesh = plsc.VectorSubcoreMesh(core_axis_name="c", subcore_axis_name="s")

    @functools.partial(
        pl.kernel,
        mesh=mesh,
        out_type=jax.ShapeDtypeStruct((n_tokens * n_per * EMBEDDING_DIM,),
                                      jnp.float32),
        scratch_types=[
            pltpu.VMEM((n_per, i_per_w), jnp.int32),
            pltpu.VMEM((2, IBLK, EMBEDDING_DIM), jnp.float32),
            pltpu.VMEM((2, EMBEDDING_DIM * IBLK), jnp.float32),
            pltpu.SemaphoreType.DMA,
            pltpu.SemaphoreType.DMA,
        ],
        compiler_params=pltpu.CompilerParams(use_tc_tiling_on_sc=False,
                                             needs_layout_passes=False),
    )
    def lookup_kernel(tok_hbm, table_hbm, out_hbm, idx_v, rows_v, t_v,
                      gsem, wsem):
        wid = lax.axis_index("s") * _NC + lax.axis_index("c")
        i0w = wid * i_per_w
        pltpu.sync_copy(tok_hbm.at[:, pl.ds(i0w, i_per_w)], idx_v)
        lanes = lax.iota(jnp.int32, 16)

        def fire_gather(job, buf):
            j = lax.div(job, nblk)
            ib = lax.rem(job, nblk)
            pltpu.async_copy(
                table_hbm.at[idx_v.at[j, pl.ds(ib * IBLK, IBLK)]],
                rows_v.at[buf],
                gsem,
            )

        def drain_gather(buf):
            pltpu.make_async_copy(
                table_hbm.at[pl.ds(0, IBLK)],
                rows_v.at[buf],
                gsem,
            ).wait()

        def drain_writebacks():
            pltpu.make_async_copy(
                t_v.at[0],
                out_hbm.at[pl.ds(0, EMBEDDING_DIM * IBLK)],
                wsem,
            ).wait()

        def transpose_into(buf):
            n_ic = IBLK // 16

            @plsc.parallel_loop(0, EMBEDDING_DIM * n_ic, 1, unroll=8)
            def _(q):
                d = lax.div(q, n_ic)
                ic = lax.rem(q, n_ic)
                vals = plsc.load_gather(
                    rows_v.at[buf],
                    [ic * 16 + lanes, jnp.full((16,), 0, jnp.int32) + d],
                )
                t_v[buf, pl.ds(d * IBLK + ic * 16, 16)] = vals

        def fire_writebacks(job, buf):
            j = lax.div(job, nblk)
            ib = lax.rem(job, nblk)
            gblk = wid * nblk + ib
            for db in range(dtiles):
                pltpu.async_copy(
                    t_v.at[buf, pl.ds(db * TILE, TILE)],
                    out_hbm.at[pl.ds(((j * dtiles + db) * n_iblk + gblk)
                                     * TILE, TILE)],
                    wsem,
                )

        fire_gather(0, 0)
        fire_writebacks(0, 1)

        def job_body(job, carry):
            cur = lax.rem(job, 2)
            nxt = 1 - cur
            drain_gather(cur)
            nxt_job = lax.min(job + 1, n_jobs - 1)
            fire_gather(nxt_job, nxt)
            transpose_into(cur)
            drain_writebacks()
            fire_writebacks(job, cur)
            return carry

        lax.fori_loop(0, n_jobs, job_body, 0)
        drain_gather(lax.rem(n_jobs, 2))
        drain_writebacks()

    return lookup_kernel


def kernel(token_ids, weight):
    n_tokens, n_per = token_ids.shape
    tok2 = token_ids.T.astype(jnp.int32)
    out1d = _make_lookup(n_tokens, n_per)(tok2, weight)
    o5 = out1d.reshape(n_per, EMBEDDING_DIM // 8, n_tokens // IBLK, 8, IBLK)
    return o5.transpose(2, 4, 0, 1, 3).reshape(n_tokens, n_per, EMBEDDING_DIM)


# speedup vs baseline: 1.4495x; 1.2139x over previous
import functools

import jax
import jax.numpy as jnp
from jax import lax
from jax.experimental import pallas as pl
from jax.experimental.pallas import tpu as pltpu
from jax.experimental.pallas import tpu_sc as plsc

NUM_EMBEDDING = 1000000
EMBEDDING_DIM = 64
IBLK = 128
TILE = 8 * IBLK

_INFO = plsc.get_sparse_core_info()
_NC = _INFO.num_cores
_NS = _INFO.num_subcores
_NW = _NC * _NS


def _make_lookup(n_tokens, n_per):
    i_per_w = n_tokens // _NW
    nblk = i_per_w // IBLK
    n_iblk = n_tokens // IBLK
    n_jobs = n_per * nblk
    dtiles = EMBEDDING_DIM // 8
    mesh = plsc.VectorSubcoreMesh(core_axis_name="c", subcore_axis_name="s")

    @functools.partial(
        pl.kernel,
        mesh=mesh,
        out_type=jax.ShapeDtypeStruct((n_tokens * n_per * EMBEDDING_DIM,),
                                      jnp.float32),
        scratch_types=[
            pltpu.VMEM((n_per, i_per_w), jnp.int32),
            pltpu.VMEM((2, IBLK, EMBEDDING_DIM), jnp.float32),
            pltpu.VMEM((2, EMBEDDING_DIM * IBLK), jnp.float32),
            pltpu.SemaphoreType.DMA,
            pltpu.SemaphoreType.DMA,
        ],
        compiler_params=pltpu.CompilerParams(use_tc_tiling_on_sc=False,
                                             needs_layout_passes=False),
    )
    def lookup_kernel(tok_hbm, table_hbm, out_hbm, idx_v, rows_v, t_v,
                      gsem, wsem):
        wid = lax.axis_index("s") * _NC + lax.axis_index("c")
        i0w = wid * i_per_w
        pltpu.sync_copy(tok_hbm.at[:, pl.ds(i0w, i_per_w)], idx_v)
        lanes = lax.iota(jnp.int32, 16)

        def fire_gather(job, buf):
            j = lax.div(job, nblk)
            ib = lax.rem(job, nblk)
            pltpu.async_copy(
                table_hbm.at[idx_v.at[j, pl.ds(ib * IBLK, IBLK)]],
                rows_v.at[buf],
                gsem,
            )

        def drain_gather(buf):
            pltpu.make_async_copy(
                table_hbm.at[pl.ds(0, IBLK)],
                rows_v.at[buf],
                gsem,
            ).wait()

        def drain_writebacks():
            pltpu.make_async_copy(
                t_v.at[0],
                out_hbm.at[pl.ds(0, EMBEDDING_DIM * IBLK)],
                wsem,
            ).wait()

        def transpose_into(buf):
            @plsc.parallel_loop(0, EMBEDDING_DIM, 1, unroll=4)
            def _(d):
                dvec = jnp.full((16,), 0, jnp.int32) + d
                base = d * IBLK
                for ic in range(IBLK // 16):
                    vals = plsc.load_gather(
                        rows_v.at[buf],
                        [ic * 16 + lanes, dvec],
                    )
                    t_v[buf, pl.ds(base + ic * 16, 16)] = vals

        def fire_writebacks(job, buf):
            j = lax.div(job, nblk)
            ib = lax.rem(job, nblk)
            gblk = wid * nblk + ib
            for db in range(dtiles):
                pltpu.async_copy(
                    t_v.at[buf, pl.ds(db * TILE, TILE)],
                    out_hbm.at[pl.ds(((j * dtiles + db) * n_iblk + gblk)
                                     * TILE, TILE)],
                    wsem,
                )

        fire_gather(0, 0)
        fire_writebacks(0, 1)

        def job_body(job, carry):
            cur = lax.rem(job, 2)
            nxt = 1 - cur
            drain_gather(cur)
            nxt_job = lax.min(job + 1, n_jobs - 1)
            fire_gather(nxt_job, nxt)
            transpose_into(cur)
            drain_writebacks()
            fire_writebacks(job, cur)
            return carry

        lax.fori_loop(0, n_jobs, job_body, 0)
        drain_gather(lax.rem(n_jobs, 2))
        drain_writebacks()

    return lookup_kernel


def kernel(token_ids, weight):
    n_tokens, n_per = token_ids.shape
    tok2 = token_ids.T.astype(jnp.int32)
    out1d = _make_lookup(n_tokens, n_per)(tok2, weight)
    o5 = out1d.reshape(n_per, EMBEDDING_DIM // 8, n_tokens // IBLK, 8, IBLK)
    return o5.transpose(2, 4, 0, 1, 3).reshape(n_tokens, n_per, EMBEDDING_DIM)
